# baseline (speedup 1.0000x reference)
import functools

import jax
import jax.numpy as jnp
from jax import lax
from jax.experimental import pallas as pl
from jax.experimental.pallas import tpu as pltpu
from jax.experimental.pallas import tpu_sc as plsc

F32 = jnp.float32
_NC = 2
_NS = 16
_NW = _NC * _NS
_L = 16
_BN = 256


def _rsqrt_nr(d):
    i = plsc.bitcast(d, jnp.int32)
    y = plsc.bitcast(jnp.int32(0x5F3759DF) - (i >> 1), F32)
    for _ in range(3):
        y = y * (1.5 - 0.5 * d * y * y)
    return y


def _build_rhat(N, E):
    EC = E // _NW
    BR = 80
    mesh = plsc.VectorSubcoreMesh(core_axis_name="c", subcore_axis_name="s")

    @functools.partial(
        pl.kernel, mesh=mesh,
        compiler_params=pltpu.CompilerParams(needs_layout_passes=False),
        out_type=[jax.ShapeDtypeStruct((E,), F32)] * 3,
        scratch_types=[
            pltpu.VMEM((N,), F32),
            pltpu.VMEM((N,), F32),
            pltpu.VMEM((N,), F32),
            pltpu.VMEM((BR,), jnp.int32),
            pltpu.VMEM((BR,), jnp.int32),
            pltpu.VMEM((BR,), F32),
            pltpu.VMEM((BR,), F32),
            pltpu.VMEM((BR,), F32),
        ],
    )
    def rhat_k(px_hbm, py_hbm, pz_hbm, src_hbm, dst_hbm,
               rh0_hbm, rh1_hbm, rh2_hbm, px_v, py_v, pz_v,
               si_v, di_v, ro0_v, ro1_v, ro2_v):
        cid = lax.axis_index("c")
        sid = lax.axis_index("s")
        wid = sid * _NC + cid
        pltpu.sync_copy(px_hbm, px_v)
        pltpu.sync_copy(py_hbm, py_v)
        pltpu.sync_copy(pz_hbm, pz_v)
        base0 = wid * EC

        def batch(b, c):
            base = base0 + b * BR
            pltpu.sync_copy(src_hbm.at[pl.ds(base, BR)], si_v)
            pltpu.sync_copy(dst_hbm.at[pl.ds(base, BR)], di_v)
            for g in range(BR // _L):
                sl = pl.ds(g * _L, _L)
                si = si_v[sl]
                di = di_v[sl]
                sx = plsc.load_gather(px_v, [si])
                sy = plsc.load_gather(py_v, [si])
                sz = plsc.load_gather(pz_v, [si])
                dx = plsc.load_gather(px_v, [di])
                dy = plsc.load_gather(py_v, [di])
                dz = plsc.load_gather(pz_v, [di])
                r0 = dx - sx
                r1 = dy - sy
                r2 = dz - sz
                d = r0 * r0 + r1 * r1 + r2 * r2
                inv = 1.0 / (d * _rsqrt_nr(d) + 1e-8)
                ro0_v[sl] = r0 * inv
                ro1_v[sl] = r1 * inv
                ro2_v[sl] = r2 * inv
            pltpu.sync_copy(ro0_v, rh0_hbm.at[pl.ds(base, BR)])
            pltpu.sync_copy(ro1_v, rh1_hbm.at[pl.ds(base, BR)])
            pltpu.sync_copy(ro2_v, rh2_hbm.at[pl.ds(base, BR)])
            return c

        lax.fori_loop(0, EC // BR, batch, 0)

    return rhat_k


def _build_agg(N_pad, E, F):
    EC = E // _NW
    B = _L
    C = 400
    NSC = EC // C
    NBC = C // B
    NPAIR = (NBC - 1) // 2
    RT = N_pad // _NS
    G = F // _L
    mesh = plsc.VectorSubcoreMesh(core_axis_name="c", subcore_axis_name="s")

    @functools.partial(
        pl.kernel, mesh=mesh,
        compiler_params=pltpu.CompilerParams(needs_layout_passes=False),
        out_type=[
            jax.ShapeDtypeStruct((2, N_pad, F), F32),
            jax.ShapeDtypeStruct((2, 3, N_pad, F), F32),
        ],
        scratch_types=[
            pltpu.VMEM_SHARED((N_pad, F), F32),
            pltpu.VMEM((C,), jnp.int32),
            pltpu.VMEM((C,), jnp.int32),
            pltpu.VMEM((C,), F32),
            pltpu.VMEM((C,), F32),
            pltpu.VMEM((C,), F32),
            pltpu.VMEM((2, B, 4 * F), F32),
            pltpu.VMEM((2, B, 2 * F), F32),
            pltpu.VMEM((2, B, F), F32),
            pltpu.SemaphoreType.DMA,
            pltpu.SemaphoreType.DMA,
            pltpu.SemaphoreType.DMA,
            pltpu.SemaphoreType.DMA,
        ],
    )
    def agg_k(ts_hbm, tv0_hbm, tv1_hbm, tv2_hbm, src_hbm, dst_hbm,
              rh0_hbm, rh1_hbm, rh2_hbm, aggs_hbm, aggv_hbm,
              acc_sh, si_c, di_c, rha_c, rhb_c, rhc_c, rsA_v, rsB_v,
              co_v, g0, g1, s0, s1):
        cid = lax.axis_index("c")
        sid = lax.axis_index("s")
        wid = sid * _NC + cid
        ebase0 = wid * EC
        zidx = jnp.zeros((_L,), jnp.int32)

        def zero_acc():
            def zrow(j, c):
                for g in range(G):
                    co_v[0, j, pl.ds(g * _L, _L)] = jnp.zeros((_L,), F32)
                return c

            lax.fori_loop(0, B, zrow, 0)
            for t in range(RT // B):
                pltpu.sync_copy(co_v.at[0],
                                acc_sh.at[pl.ds(sid * RT + t * B, B)])
            plsc.subcore_barrier()

        def computeA(slot, b):
            rv0 = rha_c[pl.ds(b * B, B)]
            rv1 = rhb_c[pl.ds(b * B, B)]
            rv2 = rhc_c[pl.ds(b * B, B)]

            def edge4(ii, c):
                j0 = ii * 4
                for d in range(4):
                    jj = j0 + d
                    idxv = jnp.full((_L,), d, jnp.int32) + j0
                    a0 = rv0.at[idxv].get(mode="promise_in_bounds")
                    a1 = rv1.at[idxv].get(mode="promise_in_bounds")
                    a2 = rv2.at[idxv].get(mode="promise_in_bounds")
                    for g in range(G):
                        sl = pl.ds(g * _L, _L)
                        t0 = rsA_v[slot, jj, sl]
                        t1 = rsA_v[slot, jj, pl.ds(F + g * _L, _L)]
                        t2 = rsA_v[slot, jj, pl.ds(2 * F + g * _L, _L)]
                        t3 = rsA_v[slot, jj, pl.ds(3 * F + g * _L, _L)]
                        co_v[slot, jj, sl] = (t0 + a0 * t1 + a1 * t2
                                              + a2 * t3)
                return c

            lax.fori_loop(0, B // 4, edge4, 0)

        def computeB(slot, b):
            rv = rha_c[pl.ds(b * B, B)]

            def edge4(ii, c):
                j0 = ii * 4
                for d in range(4):
                    jj = j0 + d
                    idxv = jnp.full((_L,), d, jnp.int32) + j0
                    a = rv.at[idxv].get(mode="promise_in_bounds")
                    for g in range(G):
                        sl = pl.ds(g * _L, _L)
                        t0 = rsB_v[slot, jj, sl]
                        t1 = rsB_v[slot, jj, pl.ds(F + g * _L, _L)]
                        co_v[slot, jj, sl] = t0 + a * t1
                return c

            lax.fori_loop(0, B // 4, edge4, 0)

        def sweep(tbl_hbm, rs_v, stage_rh, compute, flush):
            zero_acc()

            def gi(b, slot, sem):
                si_vec = si_c[pl.ds(b * B, B)]
                pltpu.async_copy(tbl_hbm.at[si_vec], rs_v.at[slot], sem)

            def gw(slot, sem):
                pltpu.make_async_copy(
                    tbl_hbm.at[zidx], rs_v.at[slot], sem).wait()

            def sct(b, slot, sem):
                di_vec = di_c[pl.ds(b * B, B)]
                pltpu.async_copy(co_v.at[slot], acc_sh.at[di_vec], sem,
                                 add=True)

            def scw(slot, sem):
                pltpu.make_async_copy(
                    co_v.at[slot], acc_sh.at[zidx], sem).wait()

            def schunk(sidx, c):
                cbase = ebase0 + sidx * C
                pltpu.sync_copy(src_hbm.at[pl.ds(cbase, C)], si_c)
                pltpu.sync_copy(dst_hbm.at[pl.ds(cbase, C)], di_c)
                stage_rh(cbase)
                gi(0, 0, g0)

                def pair(p, c2):
                    b0 = 2 * p
                    gw(0, g0)
                    gi(b0 + 1, 1, g1)

                    @pl.when(p > 0)
                    def _():
                        scw(0, s0)

                    compute(0, b0)
                    sct(b0, 0, s0)
                    gw(1, g1)
                    gi(b0 + 2, 0, g0)

                    @pl.when(p > 0)
                    def _():
                        scw(1, s1)

                    compute(1, b0 + 1)
                    sct(b0 + 1, 1, s1)
                    return c2

                lax.fori_loop(0, NPAIR, pair, 0)
                gw(0, g0)
                scw(0, s0)
                compute(0, NBC - 1)
                sct(NBC - 1, 0, s0)
                scw(0, s0)
                scw(1, s1)
                return c

            lax.fori_loop(0, NSC, schunk, 0)
            plsc.subcore_barrier()
            flush()
            plsc.subcore_barrier()

        def stageA(cbase):
            pltpu.sync_copy(rh0_hbm.at[pl.ds(cbase, C)], rha_c)
            pltpu.sync_copy(rh1_hbm.at[pl.ds(cbase, C)], rhb_c)
            pltpu.sync_copy(rh2_hbm.at[pl.ds(cbase, C)], rhc_c)

        def flushA():
            pltpu.sync_copy(acc_sh.at[pl.ds(sid * RT, RT)],
                            aggs_hbm.at[cid, pl.ds(sid * RT, RT)])

        sweep(ts_hbm, rsA_v, stageA, computeA, flushA)

        for k, (tv_hbm, rh_hbm) in enumerate(
                ((tv0_hbm, rh0_hbm), (tv1_hbm, rh1_hbm), (tv2_hbm, rh2_hbm))):
            def stageB(cbase, _rh=rh_hbm):
                pltpu.sync_copy(_rh.at[pl.ds(cbase, C)], rha_c)

            def flushB(_k=k):
                pltpu.sync_copy(acc_sh.at[pl.ds(sid * RT, RT)],
                                aggv_hbm.at[cid, _k, pl.ds(sid * RT, RT)])

            sweep(tv_hbm, rsB_v, stageB, computeB, flushB)

    return agg_k


def _tables_body(s_ref, v0_ref, v1_ref, v2_ref, wsn_ref, wvs_ref, wvn_ref,
                 wsv_ref, ts_ref, tv0_ref, tv1_ref, tv2_ref):
    s = s_ref[...]
    F = s.shape[1]
    ts_ref[:, :F] = jnp.dot(s, wsn_ref[...], preferred_element_type=F32)
    a_sv = jnp.dot(s, wsv_ref[...], preferred_element_type=F32)
    wvs = wvs_ref[...]
    wvn = wvn_ref[...]
    for k, (v_ref, tv_ref) in enumerate(((v0_ref, tv0_ref), (v1_ref, tv1_ref),
                                         (v2_ref, tv2_ref))):
        vk = v_ref[...]
        ts_ref[:, (1 + k) * F:(2 + k) * F] = jnp.dot(
            vk, wvs, preferred_element_type=F32)
        tv_ref[:, :F] = jnp.dot(vk, wvn, preferred_element_type=F32)
        tv_ref[:, F:] = a_sv


def _build_tables_call(N_pad, F):
    g = N_pad // _BN
    row = pl.BlockSpec((_BN, F), lambda i: (i, 0))
    w = pl.BlockSpec((F, F), lambda i: (0, 0))
    return pl.pallas_call(
        _tables_body,
        grid=(g,),
        in_specs=[row, row, row, row, w, w, w, w],
        out_specs=[
            pl.BlockSpec((_BN, 4 * F), lambda i: (i, 0)),
            pl.BlockSpec((_BN, 2 * F), lambda i: (i, 0)),
            pl.BlockSpec((_BN, 2 * F), lambda i: (i, 0)),
            pl.BlockSpec((_BN, 2 * F), lambda i: (i, 0)),
        ],
        out_shape=[
            jax.ShapeDtypeStruct((N_pad, 4 * F), F32),
            jax.ShapeDtypeStruct((N_pad, 2 * F), F32),
            jax.ShapeDtypeStruct((N_pad, 2 * F), F32),
            jax.ShapeDtypeStruct((N_pad, 2 * F), F32),
        ],
    )


def _update_body(s_ref, v0_ref, v1_ref, v2_ref, aggs_ref, aggv_ref, wss_ref,
                 wvv_ref, so_ref, vo0_ref, vo1_ref, vo2_ref):
    s = s_ref[...]
    aggs = aggs_ref[0] + aggs_ref[1]
    so_ref[...] = jax.nn.relu(
        jnp.dot(s, wss_ref[...], preferred_element_type=F32) + aggs) + s
    wvv = wvv_ref[...]
    for k, (v_ref, vo_ref) in enumerate(((v0_ref, vo0_ref), (v1_ref, vo1_ref),
                                         (v2_ref, vo2_ref))):
        vk = v_ref[...]
        aggv = aggv_ref[0, k] + aggv_ref[1, k]
        vo_ref[...] = jnp.dot(vk, wvv, preferred_element_type=F32) + aggv + vk


def _build_update_call(N_pad, F):
    g = N_pad // _BN
    row = pl.BlockSpec((_BN, F), lambda i: (i, 0))
    w = pl.BlockSpec((F, F), lambda i: (0, 0))
    return pl.pallas_call(
        _update_body,
        grid=(g,),
        in_specs=[
            row, row, row, row,
            pl.BlockSpec((2, _BN, F), lambda i: (0, i, 0)),
            pl.BlockSpec((2, 3, _BN, F), lambda i: (0, 0, i, 0)),
            w, w,
        ],
        out_specs=[row, row, row, row],
        out_shape=[jax.ShapeDtypeStruct((N_pad, F), F32)] * 4,
    )


def _readout_body(s_ref, v0_ref, v1_ref, v2_ref, wsw_ref, wsb_ref, wvw_ref,
                  wvb_ref, sr_ref, vr_ref):
    i = pl.program_id(0)

    @pl.when(i == 0)
    def _init():
        sr_ref[...] = jnp.zeros_like(sr_ref)
        vr_ref[...] = jnp.zeros_like(vr_ref)

    s = s_ref[...]
    ws = jax.nn.sigmoid(
        jnp.sum(s * wsw_ref[...], axis=1, keepdims=True) + wsb_ref[0, 0])
    sr_ref[...] += jnp.sum(ws * s, axis=0, keepdims=True)
    for k, v_ref in enumerate((v0_ref, v1_ref, v2_ref)):
        vk = v_ref[...]
        wv = jax.nn.sigmoid(
            jnp.sum(vk * wvw_ref[...], axis=1, keepdims=True) + wvb_ref[0, 0])
        vr_ref[k:k + 1, :] += jnp.sum(wv * vk, axis=0, keepdims=True)


def _build_readout_call(N_pad, F):
    g = N_pad // _BN
    row = pl.BlockSpec((_BN, F), lambda i: (i, 0))
    vec = pl.BlockSpec((1, F), lambda i: (0, 0))
    one = pl.BlockSpec((1, 1), lambda i: (0, 0))
    return pl.pallas_call(
        _readout_body,
        grid=(g,),
        in_specs=[row, row, row, row, vec, one, vec, one],
        out_specs=[
            pl.BlockSpec((1, F), lambda i: (0, 0)),
            pl.BlockSpec((3, F), lambda i: (0, 0)),
        ],
        out_shape=[
            jax.ShapeDtypeStruct((1, F), F32),
            jax.ShapeDtypeStruct((3, F), F32),
        ],
    )


def _mlp_body(sr_ref, vr_ref, l1w_ref, l1b_ref, l2w_ref, l2b_ref, m1w_ref,
              m1b_ref, m2w_ref, m2b_ref, ows_ref, owv_ref, ob_ref, out_ref):
    h1 = jax.nn.relu(
        jnp.dot(sr_ref[...], l1w_ref[...], preferred_element_type=F32)
        + l1b_ref[...])
    hs = jnp.dot(h1, l2w_ref[...], preferred_element_type=F32) + l2b_ref[...]
    g1 = jax.nn.relu(
        jnp.dot(vr_ref[...], m1w_ref[...], preferred_element_type=F32)
        + m1b_ref[...])
    hv = jnp.dot(g1, m2w_ref[...], preferred_element_type=F32) + m2b_ref[...]
    tot = jnp.dot(hs, ows_ref[...], preferred_element_type=F32)
    out_ref[...] = tot + jnp.sum(hv * owv_ref[...]) + ob_ref[...]


def _build_mlp_call(shapes):
    full = [pl.BlockSpec(s, lambda i: tuple(0 for _ in s)) for s in shapes]
    return pl.pallas_call(
        _mlp_body,
        grid=(1,),
        in_specs=full,
        out_specs=pl.BlockSpec((1, 1), lambda i: (0, 0)),
        out_shape=jax.ShapeDtypeStruct((1, 1), F32),
    )


def kernel(x, v, pos, params, edge_index):
    N, F = x.shape
    E = edge_index.shape[1]
    N_pad = ((N + _BN - 1) // _BN) * _BN
    pad = N_pad - N

    s = jnp.pad(x, ((0, pad), (0, 0)))
    vt = jnp.pad(jnp.transpose(v, (1, 0, 2)), ((0, 0), (0, pad), (0, 0)))
    v0, v1, v2 = vt[0], vt[1], vt[2]
    src = edge_index[0]
    dst = edge_index[1]

    rh0, rh1, rh2 = _build_rhat(N, E)(
        pos[:, 0], pos[:, 1], pos[:, 2], src, dst)

    tables_call = _build_tables_call(N_pad, F)
    agg_call = _build_agg(N_pad, E, F)
    update_call = _build_update_call(N_pad, F)

    for p in params['blocks']:
        ts, tv0, tv1, tv2 = tables_call(
            s, v0, v1, v2, p['W_sn'], p['W_vs'], p['W_vn'], p['W_sv'])
        aggs, aggv = agg_call(ts, tv0, tv1, tv2, src, dst, rh0, rh1, rh2)
        s, v0, v1, v2 = update_call(
            s, v0, v1, v2, aggs, aggv, p['W_ss'], p['W_vv'])

    sr, vr = _build_readout_call(N_pad, F)(
        s, v0, v1, v2,
        params['ws_w'].reshape(1, F), params['ws_b'].reshape(1, 1),
        params['wv_w'].reshape(1, F), params['wv_b'].reshape(1, 1))

    H3 = params['s_l2_w'].shape[1]
    H = H3 // 3
    ows = params['out_w'][:H3]
    owv = params['out_w'][H3:, 0].reshape(3, H)
    mlp_in = [
        sr, vr,
        params['s_l1_w'], params['s_l1_b'].reshape(1, -1),
        params['s_l2_w'], params['s_l2_b'].reshape(1, -1),
        params['v_l1_w'], params['v_l1_b'].reshape(1, -1),
        params['v_l2_w'], params['v_l2_b'].reshape(1, -1),
        ows, owv, params['out_b'].reshape(1, 1),
    ]
    out = _build_mlp_call([a.shape for a in mlp_in])(*mlp_in)
    return out


# speedup vs baseline: 1.0277x; 1.0277x over previous
import functools

import jax
import jax.numpy as jnp
from jax import lax
from jax.experimental import pallas as pl
from jax.experimental.pallas import tpu as pltpu
from jax.experimental.pallas import tpu_sc as plsc

F32 = jnp.float32
_NC = 2
_NS = 16
_NW = _NC * _NS
_L = 16
_BN = 256


def _rsqrt_nr(d):
    i = plsc.bitcast(d, jnp.int32)
    y = plsc.bitcast(jnp.int32(0x5F3759DF) - (i >> 1), F32)
    for _ in range(3):
        y = y * (1.5 - 0.5 * d * y * y)
    return y


def _build_rhat(N, E):
    EC = E // _NW
    BR = 80
    mesh = plsc.VectorSubcoreMesh(core_axis_name="c", subcore_axis_name="s")

    @functools.partial(
        pl.kernel, mesh=mesh,
        compiler_params=pltpu.CompilerParams(needs_layout_passes=False),
        out_type=[jax.ShapeDtypeStruct((E,), F32)] * 3,
        scratch_types=[
            pltpu.VMEM((N,), F32),
            pltpu.VMEM((N,), F32),
            pltpu.VMEM((N,), F32),
            pltpu.VMEM((BR,), jnp.int32),
            pltpu.VMEM((BR,), jnp.int32),
            pltpu.VMEM((BR,), F32),
            pltpu.VMEM((BR,), F32),
            pltpu.VMEM((BR,), F32),
        ],
    )
    def rhat_k(px_hbm, py_hbm, pz_hbm, src_hbm, dst_hbm,
               rh0_hbm, rh1_hbm, rh2_hbm, px_v, py_v, pz_v,
               si_v, di_v, ro0_v, ro1_v, ro2_v):
        cid = lax.axis_index("c")
        sid = lax.axis_index("s")
        wid = sid * _NC + cid
        pltpu.sync_copy(px_hbm, px_v)
        pltpu.sync_copy(py_hbm, py_v)
        pltpu.sync_copy(pz_hbm, pz_v)
        base0 = wid * EC

        def batch(b, c):
            base = base0 + b * BR
            pltpu.sync_copy(src_hbm.at[pl.ds(base, BR)], si_v)
            pltpu.sync_copy(dst_hbm.at[pl.ds(base, BR)], di_v)
            for g in range(BR // _L):
                sl = pl.ds(g * _L, _L)
                si = si_v[sl]
                di = di_v[sl]
                sx = plsc.load_gather(px_v, [si])
                sy = plsc.load_gather(py_v, [si])
                sz = plsc.load_gather(pz_v, [si])
                dx = plsc.load_gather(px_v, [di])
                dy = plsc.load_gather(py_v, [di])
                dz = plsc.load_gather(pz_v, [di])
                r0 = dx - sx
                r1 = dy - sy
                r2 = dz - sz
                d = r0 * r0 + r1 * r1 + r2 * r2
                inv = 1.0 / (d * _rsqrt_nr(d) + 1e-8)
                ro0_v[sl] = r0 * inv
                ro1_v[sl] = r1 * inv
                ro2_v[sl] = r2 * inv
            pltpu.sync_copy(ro0_v, rh0_hbm.at[pl.ds(base, BR)])
            pltpu.sync_copy(ro1_v, rh1_hbm.at[pl.ds(base, BR)])
            pltpu.sync_copy(ro2_v, rh2_hbm.at[pl.ds(base, BR)])
            return c

        lax.fori_loop(0, EC // BR, batch, 0)

    return rhat_k


def _build_agg(N_pad, E, F):
    EC = E // _NW
    B = _L
    C = 400
    NSC = EC // C
    NBC = C // B
    NTRI = (NBC - 1) // 3
    RT = N_pad // _NS
    G = F // _L
    mesh = plsc.VectorSubcoreMesh(core_axis_name="c", subcore_axis_name="s")

    @functools.partial(
        pl.kernel, mesh=mesh,
        compiler_params=pltpu.CompilerParams(needs_layout_passes=False),
        out_type=[
            jax.ShapeDtypeStruct((2, N_pad, F), F32),
            jax.ShapeDtypeStruct((2, 3, N_pad, F), F32),
        ],
        scratch_types=[
            pltpu.VMEM_SHARED((N_pad, F), F32),
            pltpu.VMEM((C,), jnp.int32),
            pltpu.VMEM((C,), jnp.int32),
            pltpu.VMEM((C,), F32),
            pltpu.VMEM((C,), F32),
            pltpu.VMEM((C,), F32),
            pltpu.VMEM((3, B, 4 * F), F32),
            pltpu.VMEM((3, B, 2 * F), F32),
            pltpu.VMEM((3, B, F), F32),
            pltpu.SemaphoreType.DMA,
            pltpu.SemaphoreType.DMA,
            pltpu.SemaphoreType.DMA,
            pltpu.SemaphoreType.DMA,
            pltpu.SemaphoreType.DMA,
            pltpu.SemaphoreType.DMA,
        ],
    )
    def agg_k(ts_hbm, tv0_hbm, tv1_hbm, tv2_hbm, src_hbm, dst_hbm,
              rh0_hbm, rh1_hbm, rh2_hbm, aggs_hbm, aggv_hbm,
              acc_sh, si_c, di_c, rha_c, rhb_c, rhc_c, rsA_v, rsB_v,
              co_v, g0, g1, g2, s0, s1, s2):
        cid = lax.axis_index("c")
        sid = lax.axis_index("s")
        wid = sid * _NC + cid
        ebase0 = wid * EC
        zidx = jnp.zeros((_L,), jnp.int32)

        def zero_acc():
            def zrow(j, c):
                for g in range(G):
                    co_v[0, j, pl.ds(g * _L, _L)] = jnp.zeros((_L,), F32)
                return c

            lax.fori_loop(0, B, zrow, 0)
            for t in range(RT // B):
                pltpu.sync_copy(co_v.at[0],
                                acc_sh.at[pl.ds(sid * RT + t * B, B)])
            plsc.subcore_barrier()

        def computeA(slot, b):
            rv0 = rha_c[pl.ds(b * B, B)]
            rv1 = rhb_c[pl.ds(b * B, B)]
            rv2 = rhc_c[pl.ds(b * B, B)]

            def edge4(ii, c):
                j0 = ii * 4
                for d in range(4):
                    jj = j0 + d
                    idxv = jnp.full((_L,), d, jnp.int32) + j0
                    a0 = rv0.at[idxv].get(mode="promise_in_bounds")
                    a1 = rv1.at[idxv].get(mode="promise_in_bounds")
                    a2 = rv2.at[idxv].get(mode="promise_in_bounds")
                    for g in range(G):
                        sl = pl.ds(g * _L, _L)
                        t0 = rsA_v[slot, jj, sl]
                        t1 = rsA_v[slot, jj, pl.ds(F + g * _L, _L)]
                        t2 = rsA_v[slot, jj, pl.ds(2 * F + g * _L, _L)]
                        t3 = rsA_v[slot, jj, pl.ds(3 * F + g * _L, _L)]
                        co_v[slot, jj, sl] = (t0 + a0 * t1 + a1 * t2
                                              + a2 * t3)
                return c

            lax.fori_loop(0, B // 4, edge4, 0)

        def computeB(slot, b):
            rv = rha_c[pl.ds(b * B, B)]

            def edge4(ii, c):
                j0 = ii * 4
                for d in range(4):
                    jj = j0 + d
                    idxv = jnp.full((_L,), d, jnp.int32) + j0
                    a = rv.at[idxv].get(mode="promise_in_bounds")
                    for g in range(G):
                        sl = pl.ds(g * _L, _L)
                        t0 = rsB_v[slot, jj, sl]
                        t1 = rsB_v[slot, jj, pl.ds(F + g * _L, _L)]
                        co_v[slot, jj, sl] = t0 + a * t1
                return c

            lax.fori_loop(0, B // 4, edge4, 0)

        def sweep(tbl_hbm, rs_v, stage_rh, compute, flush):
            zero_acc()

            def gi(b, slot, sem):
                si_vec = si_c[pl.ds(b * B, B)]
                pltpu.async_copy(tbl_hbm.at[si_vec], rs_v.at[slot], sem)

            def gw(slot, sem):
                pltpu.make_async_copy(
                    tbl_hbm.at[zidx], rs_v.at[slot], sem).wait()

            def sct(b, slot, sem):
                di_vec = di_c[pl.ds(b * B, B)]
                pltpu.async_copy(co_v.at[slot], acc_sh.at[di_vec], sem,
                                 add=True)

            def scw(slot, sem):
                pltpu.make_async_copy(
                    co_v.at[slot], acc_sh.at[zidx], sem).wait()

            gsems = (g0, g1, g2)
            ssems = (s0, s1, s2)

            def schunk(sidx, c):
                cbase = ebase0 + sidx * C
                pltpu.sync_copy(src_hbm.at[pl.ds(cbase, C)], si_c)
                pltpu.sync_copy(dst_hbm.at[pl.ds(cbase, C)], di_c)
                stage_rh(cbase)
                gi(0, 0, g0)
                gi(1, 1, g1)

                def triple(t, c2):
                    p0 = 3 * t
                    for d in range(3):
                        p = p0 + d
                        gw(d, gsems[d])

                        @pl.when(p + 2 <= NBC - 1)
                        def _(_p=p, _d=d):
                            gi(_p + 2, (_d + 2) % 3, gsems[(_d + 2) % 3])

                        @pl.when(p >= 3)
                        def _(_d=d):
                            scw(_d, ssems[_d])

                        compute(d, p)
                        sct(p, d, ssems[d])
                    return c2

                lax.fori_loop(0, NTRI, triple, 0)
                gw(0, g0)
                scw(0, s0)
                compute(0, NBC - 1)
                sct(NBC - 1, 0, s0)
                scw(0, s0)
                scw(1, s1)
                scw(2, s2)
                return c

            lax.fori_loop(0, NSC, schunk, 0)
            plsc.subcore_barrier()
            flush()
            plsc.subcore_barrier()

        def stageA(cbase):
            pltpu.sync_copy(rh0_hbm.at[pl.ds(cbase, C)], rha_c)
            pltpu.sync_copy(rh1_hbm.at[pl.ds(cbase, C)], rhb_c)
            pltpu.sync_copy(rh2_hbm.at[pl.ds(cbase, C)], rhc_c)

        def flushA():
            pltpu.sync_copy(acc_sh.at[pl.ds(sid * RT, RT)],
                            aggs_hbm.at[cid, pl.ds(sid * RT, RT)])

        sweep(ts_hbm, rsA_v, stageA, computeA, flushA)

        for k, (tv_hbm, rh_hbm) in enumerate(
                ((tv0_hbm, rh0_hbm), (tv1_hbm, rh1_hbm), (tv2_hbm, rh2_hbm))):
            def stageB(cbase, _rh=rh_hbm):
                pltpu.sync_copy(_rh.at[pl.ds(cbase, C)], rha_c)

            def flushB(_k=k):
                pltpu.sync_copy(acc_sh.at[pl.ds(sid * RT, RT)],
                                aggv_hbm.at[cid, _k, pl.ds(sid * RT, RT)])

            sweep(tv_hbm, rsB_v, stageB, computeB, flushB)

    return agg_k


def _tables_body(s_ref, v0_ref, v1_ref, v2_ref, wsn_ref, wvs_ref, wvn_ref,
                 wsv_ref, ts_ref, tv0_ref, tv1_ref, tv2_ref):
    s = s_ref[...]
    F = s.shape[1]
    ts_ref[:, :F] = jnp.dot(s, wsn_ref[...], preferred_element_type=F32)
    a_sv = jnp.dot(s, wsv_ref[...], preferred_element_type=F32)
    wvs = wvs_ref[...]
    wvn = wvn_ref[...]
    for k, (v_ref, tv_ref) in enumerate(((v0_ref, tv0_ref), (v1_ref, tv1_ref),
                                         (v2_ref, tv2_ref))):
        vk = v_ref[...]
        ts_ref[:, (1 + k) * F:(2 + k) * F] = jnp.dot(
            vk, wvs, preferred_element_type=F32)
        tv_ref[:, :F] = jnp.dot(vk, wvn, preferred_element_type=F32)
        tv_ref[:, F:] = a_sv


def _build_tables_call(N_pad, F):
    g = N_pad // _BN
    row = pl.BlockSpec((_BN, F), lambda i: (i, 0))
    w = pl.BlockSpec((F, F), lambda i: (0, 0))
    return pl.pallas_call(
        _tables_body,
        grid=(g,),
        in_specs=[row, row, row, row, w, w, w, w],
        out_specs=[
            pl.BlockSpec((_BN, 4 * F), lambda i: (i, 0)),
            pl.BlockSpec((_BN, 2 * F), lambda i: (i, 0)),
            pl.BlockSpec((_BN, 2 * F), lambda i: (i, 0)),
            pl.BlockSpec((_BN, 2 * F), lambda i: (i, 0)),
        ],
        out_shape=[
            jax.ShapeDtypeStruct((N_pad, 4 * F), F32),
            jax.ShapeDtypeStruct((N_pad, 2 * F), F32),
            jax.ShapeDtypeStruct((N_pad, 2 * F), F32),
            jax.ShapeDtypeStruct((N_pad, 2 * F), F32),
        ],
    )


def _update_body(s_ref, v0_ref, v1_ref, v2_ref, aggs_ref, aggv_ref, wss_ref,
                 wvv_ref, so_ref, vo0_ref, vo1_ref, vo2_ref):
    s = s_ref[...]
    aggs = aggs_ref[0] + aggs_ref[1]
    so_ref[...] = jax.nn.relu(
        jnp.dot(s, wss_ref[...], preferred_element_type=F32) + aggs) + s
    wvv = wvv_ref[...]
    for k, (v_ref, vo_ref) in enumerate(((v0_ref, vo0_ref), (v1_ref, vo1_ref),
                                         (v2_ref, vo2_ref))):
        vk = v_ref[...]
        aggv = aggv_ref[0, k] + aggv_ref[1, k]
        vo_ref[...] = jnp.dot(vk, wvv, preferred_element_type=F32) + aggv + vk


def _build_update_call(N_pad, F):
    g = N_pad // _BN
    row = pl.BlockSpec((_BN, F), lambda i: (i, 0))
    w = pl.BlockSpec((F, F), lambda i: (0, 0))
    return pl.pallas_call(
        _update_body,
        grid=(g,),
        in_specs=[
            row, row, row, row,
            pl.BlockSpec((2, _BN, F), lambda i: (0, i, 0)),
            pl.BlockSpec((2, 3, _BN, F), lambda i: (0, 0, i, 0)),
            w, w,
        ],
        out_specs=[row, row, row, row],
        out_shape=[jax.ShapeDtypeStruct((N_pad, F), F32)] * 4,
    )


def _readout_body(s_ref, v0_ref, v1_ref, v2_ref, wsw_ref, wsb_ref, wvw_ref,
                  wvb_ref, sr_ref, vr_ref):
    i = pl.program_id(0)

    @pl.when(i == 0)
    def _init():
        sr_ref[...] = jnp.zeros_like(sr_ref)
        vr_ref[...] = jnp.zeros_like(vr_ref)

    s = s_ref[...]
    ws = jax.nn.sigmoid(
        jnp.sum(s * wsw_ref[...], axis=1, keepdims=True) + wsb_ref[0, 0])
    sr_ref[...] += jnp.sum(ws * s, axis=0, keepdims=True)
    for k, v_ref in enumerate((v0_ref, v1_ref, v2_ref)):
        vk = v_ref[...]
        wv = jax.nn.sigmoid(
            jnp.sum(vk * wvw_ref[...], axis=1, keepdims=True) + wvb_ref[0, 0])
        vr_ref[k:k + 1, :] += jnp.sum(wv * vk, axis=0, keepdims=True)


def _build_readout_call(N_pad, F):
    g = N_pad // _BN
    row = pl.BlockSpec((_BN, F), lambda i: (i, 0))
    vec = pl.BlockSpec((1, F), lambda i: (0, 0))
    one = pl.BlockSpec((1, 1), lambda i: (0, 0))
    return pl.pallas_call(
        _readout_body,
        grid=(g,),
        in_specs=[row, row, row, row, vec, one, vec, one],
        out_specs=[
            pl.BlockSpec((1, F), lambda i: (0, 0)),
            pl.BlockSpec((3, F), lambda i: (0, 0)),
        ],
        out_shape=[
            jax.ShapeDtypeStruct((1, F), F32),
            jax.ShapeDtypeStruct((3, F), F32),
        ],
    )


def _mlp_body(sr_ref, vr_ref, l1w_ref, l1b_ref, l2w_ref, l2b_ref, m1w_ref,
              m1b_ref, m2w_ref, m2b_ref, ows_ref, owv_ref, ob_ref, out_ref):
    h1 = jax.nn.relu(
        jnp.dot(sr_ref[...], l1w_ref[...], preferred_element_type=F32)
        + l1b_ref[...])
    hs = jnp.dot(h1, l2w_ref[...], preferred_element_type=F32) + l2b_ref[...]
    g1 = jax.nn.relu(
        jnp.dot(vr_ref[...], m1w_ref[...], preferred_element_type=F32)
        + m1b_ref[...])
    hv = jnp.dot(g1, m2w_ref[...], preferred_element_type=F32) + m2b_ref[...]
    tot = jnp.dot(hs, ows_ref[...], preferred_element_type=F32)
    out_ref[...] = tot + jnp.sum(hv * owv_ref[...]) + ob_ref[...]


def _build_mlp_call(shapes):
    full = [pl.BlockSpec(s, lambda i: tuple(0 for _ in s)) for s in shapes]
    return pl.pallas_call(
        _mlp_body,
        grid=(1,),
        in_specs=full,
        out_specs=pl.BlockSpec((1, 1), lambda i: (0, 0)),
        out_shape=jax.ShapeDtypeStruct((1, 1), F32),
    )


def kernel(x, v, pos, params, edge_index):
    N, F = x.shape
    E = edge_index.shape[1]
    N_pad = ((N + _BN - 1) // _BN) * _BN
    pad = N_pad - N

    s = jnp.pad(x, ((0, pad), (0, 0)))
    vt = jnp.pad(jnp.transpose(v, (1, 0, 2)), ((0, 0), (0, pad), (0, 0)))
    v0, v1, v2 = vt[0], vt[1], vt[2]
    src = edge_index[0]
    dst = edge_index[1]

    rh0, rh1, rh2 = _build_rhat(N, E)(
        pos[:, 0], pos[:, 1], pos[:, 2], src, dst)

    tables_call = _build_tables_call(N_pad, F)
    agg_call = _build_agg(N_pad, E, F)
    update_call = _build_update_call(N_pad, F)

    for p in params['blocks']:
        ts, tv0, tv1, tv2 = tables_call(
            s, v0, v1, v2, p['W_sn'], p['W_vs'], p['W_vn'], p['W_sv'])
        aggs, aggv = agg_call(ts, tv0, tv1, tv2, src, dst, rh0, rh1, rh2)
        s, v0, v1, v2 = update_call(
            s, v0, v1, v2, aggs, aggv, p['W_ss'], p['W_vv'])

    sr, vr = _build_readout_call(N_pad, F)(
        s, v0, v1, v2,
        params['ws_w'].reshape(1, F), params['ws_b'].reshape(1, 1),
        params['wv_w'].reshape(1, F), params['wv_b'].reshape(1, 1))

    H3 = params['s_l2_w'].shape[1]
    H = H3 // 3
    ows = params['out_w'][:H3]
    owv = params['out_w'][H3:, 0].reshape(3, H)
    mlp_in = [
        sr, vr,
        params['s_l1_w'], params['s_l1_b'].reshape(1, -1),
        params['s_l2_w'], params['s_l2_b'].reshape(1, -1),
        params['v_l1_w'], params['v_l1_b'].reshape(1, -1),
        params['v_l2_w'], params['v_l2_b'].reshape(1, -1),
        ows, owv, params['out_b'].reshape(1, 1),
    ]
    out = _build_mlp_call([a.shape for a in mlp_in])(*mlp_in)
    return out


# speedup vs baseline: 1.1006x; 1.0709x over previous
import functools

import jax
import jax.numpy as jnp
from jax import lax
from jax.experimental import pallas as pl
from jax.experimental.pallas import tpu as pltpu
from jax.experimental.pallas import tpu_sc as plsc

F32 = jnp.float32
_NC = 2
_NS = 16
_NW = _NC * _NS
_L = 16
_BN = 256


def _rsqrt_nr(d):
    i = plsc.bitcast(d, jnp.int32)
    y = plsc.bitcast(jnp.int32(0x5F3759DF) - (i >> 1), F32)
    for _ in range(3):
        y = y * (1.5 - 0.5 * d * y * y)
    return y


def _build_rhat(N, E):
    EC = E // _NW
    BR = 80
    mesh = plsc.VectorSubcoreMesh(core_axis_name="c", subcore_axis_name="s")

    @functools.partial(
        pl.kernel, mesh=mesh,
        compiler_params=pltpu.CompilerParams(needs_layout_passes=False),
        out_type=[jax.ShapeDtypeStruct((E,), F32)] * 3,
        scratch_types=[
            pltpu.VMEM((N,), F32),
            pltpu.VMEM((N,), F32),
            pltpu.VMEM((N,), F32),
            pltpu.VMEM((BR,), jnp.int32),
            pltpu.VMEM((BR,), jnp.int32),
            pltpu.VMEM((BR,), F32),
            pltpu.VMEM((BR,), F32),
            pltpu.VMEM((BR,), F32),
        ],
    )
    def rhat_k(px_hbm, py_hbm, pz_hbm, src_hbm, dst_hbm,
               rh0_hbm, rh1_hbm, rh2_hbm, px_v, py_v, pz_v,
               si_v, di_v, ro0_v, ro1_v, ro2_v):
        cid = lax.axis_index("c")
        sid = lax.axis_index("s")
        wid = sid * _NC + cid
        pltpu.sync_copy(px_hbm, px_v)
        pltpu.sync_copy(py_hbm, py_v)
        pltpu.sync_copy(pz_hbm, pz_v)
        base0 = wid * EC

        def batch(b, c):
            base = base0 + b * BR
            pltpu.sync_copy(src_hbm.at[pl.ds(base, BR)], si_v)
            pltpu.sync_copy(dst_hbm.at[pl.ds(base, BR)], di_v)
            for g in range(BR // _L):
                sl = pl.ds(g * _L, _L)
                si = si_v[sl]
                di = di_v[sl]
                sx = plsc.load_gather(px_v, [si])
                sy = plsc.load_gather(py_v, [si])
                sz = plsc.load_gather(pz_v, [si])
                dx = plsc.load_gather(px_v, [di])
                dy = plsc.load_gather(py_v, [di])
                dz = plsc.load_gather(pz_v, [di])
                r0 = dx - sx
                r1 = dy - sy
                r2 = dz - sz
                d = r0 * r0 + r1 * r1 + r2 * r2
                inv = 1.0 / (d * _rsqrt_nr(d) + 1e-8)
                ro0_v[sl] = r0 * inv
                ro1_v[sl] = r1 * inv
                ro2_v[sl] = r2 * inv
            pltpu.sync_copy(ro0_v, rh0_hbm.at[pl.ds(base, BR)])
            pltpu.sync_copy(ro1_v, rh1_hbm.at[pl.ds(base, BR)])
            pltpu.sync_copy(ro2_v, rh2_hbm.at[pl.ds(base, BR)])
            return c

        lax.fori_loop(0, EC // BR, batch, 0)

    return rhat_k


def _build_agg(N_pad, E, F):
    EC = E // _NW
    B = _L
    C = 400
    NSC = EC // C
    NBC = C // B
    NTRI = (NBC - 1) // 3
    RT = N_pad // _NS
    G = F // _L
    mesh = plsc.VectorSubcoreMesh(core_axis_name="c", subcore_axis_name="s")

    @functools.partial(
        pl.kernel, mesh=mesh,
        compiler_params=pltpu.CompilerParams(needs_layout_passes=False),
        out_type=[
            jax.ShapeDtypeStruct((2, N_pad, F), F32),
            jax.ShapeDtypeStruct((2, 3, N_pad, F), F32),
        ],
        scratch_types=[
            pltpu.VMEM_SHARED((N_pad, F), F32),
            pltpu.VMEM((2 * C,), jnp.int32),
            pltpu.VMEM((2 * C,), jnp.int32),
            pltpu.VMEM((2 * C,), F32),
            pltpu.VMEM((2 * C,), F32),
            pltpu.VMEM((2 * C,), F32),
            pltpu.VMEM((3, B, 4 * F), F32),
            pltpu.VMEM((3, B, 2 * F), F32),
            pltpu.VMEM((3, B, F), F32),
            pltpu.SemaphoreType.DMA,
            pltpu.SemaphoreType.DMA,
            pltpu.SemaphoreType.DMA,
            pltpu.SemaphoreType.DMA,
            pltpu.SemaphoreType.DMA,
            pltpu.SemaphoreType.DMA,
            pltpu.SemaphoreType.DMA,
        ],
    )
    def agg_k(ts_hbm, tv0_hbm, tv1_hbm, tv2_hbm, src_hbm, dst_hbm,
              rh0_hbm, rh1_hbm, rh2_hbm, aggs_hbm, aggv_hbm,
              acc_sh, si_c, di_c, rha_c, rhb_c, rhc_c, rsA_v, rsB_v,
              co_v, g0, g1, g2, s0, s1, s2, m0):
        cid = lax.axis_index("c")
        sid = lax.axis_index("s")
        wid = sid * _NC + cid
        ebase0 = wid * EC
        zidx = jnp.zeros((_L,), jnp.int32)

        def zero_acc():
            def zrow(j, c):
                for g in range(G):
                    co_v[0, j, pl.ds(g * _L, _L)] = jnp.zeros((_L,), F32)
                return c

            lax.fori_loop(0, B, zrow, 0)
            for t in range(RT // B):
                pltpu.sync_copy(co_v.at[0],
                                acc_sh.at[pl.ds(sid * RT + t * B, B)])
            plsc.subcore_barrier()

        def computeA(slot, b, off):
            rv0 = rha_c[pl.ds(off + b * B, B)]
            rv1 = rhb_c[pl.ds(off + b * B, B)]
            rv2 = rhc_c[pl.ds(off + b * B, B)]

            def edge4(ii, c):
                j0 = ii * 4
                for d in range(4):
                    jj = j0 + d
                    idxv = jnp.full((_L,), d, jnp.int32) + j0
                    a0 = rv0.at[idxv].get(mode="promise_in_bounds")
                    a1 = rv1.at[idxv].get(mode="promise_in_bounds")
                    a2 = rv2.at[idxv].get(mode="promise_in_bounds")
                    for g in range(G):
                        sl = pl.ds(g * _L, _L)
                        t0 = rsA_v[slot, jj, sl]
                        t1 = rsA_v[slot, jj, pl.ds(F + g * _L, _L)]
                        t2 = rsA_v[slot, jj, pl.ds(2 * F + g * _L, _L)]
                        t3 = rsA_v[slot, jj, pl.ds(3 * F + g * _L, _L)]
                        co_v[slot, jj, sl] = (t0 + a0 * t1 + a1 * t2
                                              + a2 * t3)
                return c

            lax.fori_loop(0, B // 4, edge4, 0)

        def computeB(slot, b, off):
            rv = rha_c[pl.ds(off + b * B, B)]

            def edge4(ii, c):
                j0 = ii * 4
                for d in range(4):
                    jj = j0 + d
                    idxv = jnp.full((_L,), d, jnp.int32) + j0
                    a = rv.at[idxv].get(mode="promise_in_bounds")
                    for g in range(G):
                        sl = pl.ds(g * _L, _L)
                        t0 = rsB_v[slot, jj, sl]
                        t1 = rsB_v[slot, jj, pl.ds(F + g * _L, _L)]
                        co_v[slot, jj, sl] = t0 + a * t1
                return c

            lax.fori_loop(0, B // 4, edge4, 0)

        def sweep(tbl_hbm, rs_v, meta_issue, meta_wait, compute, flush):
            zero_acc()

            def gi(b, off, slot, sem):
                si_vec = si_c[pl.ds(off + b * B, B)]
                pltpu.async_copy(tbl_hbm.at[si_vec], rs_v.at[slot], sem)

            def gw(slot, sem):
                pltpu.make_async_copy(
                    tbl_hbm.at[zidx], rs_v.at[slot], sem).wait()

            def sct(b, off, slot, sem):
                di_vec = di_c[pl.ds(off + b * B, B)]
                pltpu.async_copy(co_v.at[slot], acc_sh.at[di_vec], sem,
                                 add=True)

            def scw(slot, sem):
                pltpu.make_async_copy(
                    co_v.at[slot], acc_sh.at[zidx], sem).wait()

            gsems = (g0, g1, g2)
            ssems = (s0, s1, s2)

            meta_issue(ebase0, 0)

            def schunk(sidx, c):
                off = lax.rem(sidx, 2) * C
                meta_wait()

                @pl.when(sidx + 1 < NSC)
                def _():
                    meta_issue(ebase0 + (sidx + 1) * C, C - off)

                gi(0, off, 0, g0)
                gi(1, off, 1, g1)

                def triple(t, c2):
                    p0 = 3 * t
                    for d in range(3):
                        p = p0 + d
                        gw(d, gsems[d])

                        @pl.when(p + 2 <= NBC - 1)
                        def _(_p=p, _d=d):
                            gi(_p + 2, off, (_d + 2) % 3,
                               gsems[(_d + 2) % 3])

                        @pl.when(p >= 3)
                        def _(_d=d):
                            scw(_d, ssems[_d])

                        compute(d, p, off)
                        sct(p, off, d, ssems[d])
                    return c2

                lax.fori_loop(0, NTRI, triple, 0)
                gw(0, g0)
                scw(0, s0)
                compute(0, NBC - 1, off)
                sct(NBC - 1, off, 0, s0)
                scw(0, s0)
                scw(1, s1)
                scw(2, s2)
                return c

            lax.fori_loop(0, NSC, schunk, 0)
            plsc.subcore_barrier()
            flush()
            plsc.subcore_barrier()

        def issueA(cbase, off):
            sl = pl.ds(cbase, C)
            dl = pl.ds(off, C)
            pltpu.async_copy(src_hbm.at[sl], si_c.at[dl], m0)
            pltpu.async_copy(dst_hbm.at[sl], di_c.at[dl], m0)
            pltpu.async_copy(rh0_hbm.at[sl], rha_c.at[dl], m0)
            pltpu.async_copy(rh1_hbm.at[sl], rhb_c.at[dl], m0)
            pltpu.async_copy(rh2_hbm.at[sl], rhc_c.at[dl], m0)

        def waitA():
            sl = pl.ds(0, C)
            pltpu.make_async_copy(src_hbm.at[sl], si_c.at[sl], m0).wait()
            pltpu.make_async_copy(dst_hbm.at[sl], di_c.at[sl], m0).wait()
            pltpu.make_async_copy(rh0_hbm.at[sl], rha_c.at[sl], m0).wait()
            pltpu.make_async_copy(rh1_hbm.at[sl], rhb_c.at[sl], m0).wait()
            pltpu.make_async_copy(rh2_hbm.at[sl], rhc_c.at[sl], m0).wait()

        def flushA():
            pltpu.sync_copy(acc_sh.at[pl.ds(sid * RT, RT)],
                            aggs_hbm.at[cid, pl.ds(sid * RT, RT)])

        sweep(ts_hbm, rsA_v, issueA, waitA, computeA, flushA)

        for k, (tv_hbm, rh_hbm) in enumerate(
                ((tv0_hbm, rh0_hbm), (tv1_hbm, rh1_hbm), (tv2_hbm, rh2_hbm))):
            def issueB(cbase, off, _rh=rh_hbm):
                sl = pl.ds(cbase, C)
                dl = pl.ds(off, C)
                pltpu.async_copy(src_hbm.at[sl], si_c.at[dl], m0)
                pltpu.async_copy(dst_hbm.at[sl], di_c.at[dl], m0)
                pltpu.async_copy(_rh.at[sl], rha_c.at[dl], m0)

            def waitB(_rh=rh_hbm):
                sl = pl.ds(0, C)
                pltpu.make_async_copy(src_hbm.at[sl], si_c.at[sl],
                                      m0).wait()
                pltpu.make_async_copy(dst_hbm.at[sl], di_c.at[sl],
                                      m0).wait()
                pltpu.make_async_copy(_rh.at[sl], rha_c.at[sl], m0).wait()

            def flushB(_k=k):
                pltpu.sync_copy(acc_sh.at[pl.ds(sid * RT, RT)],
                                aggv_hbm.at[cid, _k, pl.ds(sid * RT, RT)])

            sweep(tv_hbm, rsB_v, issueB, waitB, computeB, flushB)

    return agg_k


def _tables_body(s_ref, v0_ref, v1_ref, v2_ref, wsn_ref, wvs_ref, wvn_ref,
                 wsv_ref, ts_ref, tv0_ref, tv1_ref, tv2_ref):
    s = s_ref[...]
    F = s.shape[1]
    ts_ref[:, :F] = jnp.dot(s, wsn_ref[...], preferred_element_type=F32)
    a_sv = jnp.dot(s, wsv_ref[...], preferred_element_type=F32)
    wvs = wvs_ref[...]
    wvn = wvn_ref[...]
    for k, (v_ref, tv_ref) in enumerate(((v0_ref, tv0_ref), (v1_ref, tv1_ref),
                                         (v2_ref, tv2_ref))):
        vk = v_ref[...]
        ts_ref[:, (1 + k) * F:(2 + k) * F] = jnp.dot(
            vk, wvs, preferred_element_type=F32)
        tv_ref[:, :F] = jnp.dot(vk, wvn, preferred_element_type=F32)
        tv_ref[:, F:] = a_sv


def _build_tables_call(N_pad, F):
    g = N_pad // _BN
    row = pl.BlockSpec((_BN, F), lambda i: (i, 0))
    w = pl.BlockSpec((F, F), lambda i: (0, 0))
    return pl.pallas_call(
        _tables_body,
        grid=(g,),
        in_specs=[row, row, row, row, w, w, w, w],
        out_specs=[
            pl.BlockSpec((_BN, 4 * F), lambda i: (i, 0)),
            pl.BlockSpec((_BN, 2 * F), lambda i: (i, 0)),
            pl.BlockSpec((_BN, 2 * F), lambda i: (i, 0)),
            pl.BlockSpec((_BN, 2 * F), lambda i: (i, 0)),
        ],
        out_shape=[
            jax.ShapeDtypeStruct((N_pad, 4 * F), F32),
            jax.ShapeDtypeStruct((N_pad, 2 * F), F32),
            jax.ShapeDtypeStruct((N_pad, 2 * F), F32),
            jax.ShapeDtypeStruct((N_pad, 2 * F), F32),
        ],
    )


def _update_body(s_ref, v0_ref, v1_ref, v2_ref, aggs_ref, aggv_ref, wss_ref,
                 wvv_ref, so_ref, vo0_ref, vo1_ref, vo2_ref):
    s = s_ref[...]
    aggs = aggs_ref[0] + aggs_ref[1]
    so_ref[...] = jax.nn.relu(
        jnp.dot(s, wss_ref[...], preferred_element_type=F32) + aggs) + s
    wvv = wvv_ref[...]
    for k, (v_ref, vo_ref) in enumerate(((v0_ref, vo0_ref), (v1_ref, vo1_ref),
                                         (v2_ref, vo2_ref))):
        vk = v_ref[...]
        aggv = aggv_ref[0, k] + aggv_ref[1, k]
        vo_ref[...] = jnp.dot(vk, wvv, preferred_element_type=F32) + aggv + vk


def _build_update_call(N_pad, F):
    g = N_pad // _BN
    row = pl.BlockSpec((_BN, F), lambda i: (i, 0))
    w = pl.BlockSpec((F, F), lambda i: (0, 0))
    return pl.pallas_call(
        _update_body,
        grid=(g,),
        in_specs=[
            row, row, row, row,
            pl.BlockSpec((2, _BN, F), lambda i: (0, i, 0)),
            pl.BlockSpec((2, 3, _BN, F), lambda i: (0, 0, i, 0)),
            w, w,
        ],
        out_specs=[row, row, row, row],
        out_shape=[jax.ShapeDtypeStruct((N_pad, F), F32)] * 4,
    )


def _readout_body(s_ref, v0_ref, v1_ref, v2_ref, wsw_ref, wsb_ref, wvw_ref,
                  wvb_ref, sr_ref, vr_ref):
    i = pl.program_id(0)

    @pl.when(i == 0)
    def _init():
        sr_ref[...] = jnp.zeros_like(sr_ref)
        vr_ref[...] = jnp.zeros_like(vr_ref)

    s = s_ref[...]
    ws = jax.nn.sigmoid(
        jnp.sum(s * wsw_ref[...], axis=1, keepdims=True) + wsb_ref[0, 0])
    sr_ref[...] += jnp.sum(ws * s, axis=0, keepdims=True)
    for k, v_ref in enumerate((v0_ref, v1_ref, v2_ref)):
        vk = v_ref[...]
        wv = jax.nn.sigmoid(
            jnp.sum(vk * wvw_ref[...], axis=1, keepdims=True) + wvb_ref[0, 0])
        vr_ref[k:k + 1, :] += jnp.sum(wv * vk, axis=0, keepdims=True)


def _build_readout_call(N_pad, F):
    g = N_pad // _BN
    row = pl.BlockSpec((_BN, F), lambda i: (i, 0))
    vec = pl.BlockSpec((1, F), lambda i: (0, 0))
    one = pl.BlockSpec((1, 1), lambda i: (0, 0))
    return pl.pallas_call(
        _readout_body,
        grid=(g,),
        in_specs=[row, row, row, row, vec, one, vec, one],
        out_specs=[
            pl.BlockSpec((1, F), lambda i: (0, 0)),
            pl.BlockSpec((3, F), lambda i: (0, 0)),
        ],
        out_shape=[
            jax.ShapeDtypeStruct((1, F), F32),
            jax.ShapeDtypeStruct((3, F), F32),
        ],
    )


def _mlp_body(sr_ref, vr_ref, l1w_ref, l1b_ref, l2w_ref, l2b_ref, m1w_ref,
              m1b_ref, m2w_ref, m2b_ref, ows_ref, owv_ref, ob_ref, out_ref):
    h1 = jax.nn.relu(
        jnp.dot(sr_ref[...], l1w_ref[...], preferred_element_type=F32)
        + l1b_ref[...])
    hs = jnp.dot(h1, l2w_ref[...], preferred_element_type=F32) + l2b_ref[...]
    g1 = jax.nn.relu(
        jnp.dot(vr_ref[...], m1w_ref[...], preferred_element_type=F32)
        + m1b_ref[...])
    hv = jnp.dot(g1, m2w_ref[...], preferred_element_type=F32) + m2b_ref[...]
    tot = jnp.dot(hs, ows_ref[...], preferred_element_type=F32)
    out_ref[...] = tot + jnp.sum(hv * owv_ref[...]) + ob_ref[...]


def _build_mlp_call(shapes):
    full = [pl.BlockSpec(s, lambda i: tuple(0 for _ in s)) for s in shapes]
    return pl.pallas_call(
        _mlp_body,
        grid=(1,),
        in_specs=full,
        out_specs=pl.BlockSpec((1, 1), lambda i: (0, 0)),
        out_shape=jax.ShapeDtypeStruct((1, 1), F32),
    )


def kernel(x, v, pos, params, edge_index):
    N, F = x.shape
    E = edge_index.shape[1]
    N_pad = ((N + _BN - 1) // _BN) * _BN
    pad = N_pad - N

    s = jnp.pad(x, ((0, pad), (0, 0)))
    vt = jnp.pad(jnp.transpose(v, (1, 0, 2)), ((0, 0), (0, pad), (0, 0)))
    v0, v1, v2 = vt[0], vt[1], vt[2]
    src = edge_index[0]
    dst = edge_index[1]

    rh0, rh1, rh2 = _build_rhat(N, E)(
        pos[:, 0], pos[:, 1], pos[:, 2], src, dst)

    tables_call = _build_tables_call(N_pad, F)
    agg_call = _build_agg(N_pad, E, F)
    update_call = _build_update_call(N_pad, F)

    for p in params['blocks']:
        ts, tv0, tv1, tv2 = tables_call(
            s, v0, v1, v2, p['W_sn'], p['W_vs'], p['W_vn'], p['W_sv'])
        aggs, aggv = agg_call(ts, tv0, tv1, tv2, src, dst, rh0, rh1, rh2)
        s, v0, v1, v2 = update_call(
            s, v0, v1, v2, aggs, aggv, p['W_ss'], p['W_vv'])

    sr, vr = _build_readout_call(N_pad, F)(
        s, v0, v1, v2,
        params['ws_w'].reshape(1, F), params['ws_b'].reshape(1, 1),
        params['wv_w'].reshape(1, F), params['wv_b'].reshape(1, 1))

    H3 = params['s_l2_w'].shape[1]
    H = H3 // 3
    ows = params['out_w'][:H3]
    owv = params['out_w'][H3:, 0].reshape(3, H)
    mlp_in = [
        sr, vr,
        params['s_l1_w'], params['s_l1_b'].reshape(1, -1),
        params['s_l2_w'], params['s_l2_b'].reshape(1, -1),
        params['v_l1_w'], params['v_l1_b'].reshape(1, -1),
        params['v_l2_w'], params['v_l2_b'].reshape(1, -1),
        ows, owv, params['out_b'].reshape(1, 1),
    ]
    out = _build_mlp_call([a.shape for a in mlp_in])(*mlp_in)
    return out
